# Initial kernel scaffold; baseline (speedup 1.0000x reference)
import jax
import jax.numpy as jnp
from jax.experimental import pallas as pl


def kernel(roi_boxes3d, gt_boxes3d, rpn_xyz, rpn_features, seg_mask, pts_depth):
    raise NotImplementedError("write your pallas kernel here")


# speedup vs baseline: 12.5494x; 12.5494x over previous
import functools

import jax
import jax.numpy as jnp
import numpy as np
from jax import lax
from jax.experimental import pallas as pl
from jax.experimental.pallas import tpu as pltpu
from jax.experimental.pallas import tpu_sc as plsc

ROI_PER_IMAGE = 64
FG_PER_IMAGE = 16
BG_PER_IMAGE = 48
REG_FG_THRESH = 0.55
CLS_FG_THRESH = 0.6
CLS_BG_THRESH = 0.45
NUM_POINTS = 512
EW = 1.0

B = 4
M = 20000
MP = 20480
MR, MC = 160, 128
N = 64
P = 16384
PR, PC = 128, 128
CF = 130
NPAIR = B * ROI_PER_IMAGE


def _k1_body(rx, ry, rz, rh, rw, rl, rr, gt, selbox, selgt, seliou,
             key_s, ga_s):
    X = rx[0]
    Y = ry[0]
    Z = rz[0]
    H = rh[0]
    W = rw[0]
    L = rl[0]
    R = rr[0]
    va = H * W * L
    xl_max = X + L * 0.5
    xl_min = X - L * 0.5
    zw_max = Z + W * 0.5
    zw_min = Z - W * 0.5
    ymh = Y - H

    neg_inf = jnp.float32(-jnp.inf)

    def iou_step(j, carry):
        max_ov, gt_asgn = carry
        gx = gt[0, j, 0]
        gy = gt[0, j, 1]
        gz = gt[0, j, 2]
        gh = gt[0, j, 3]
        gw = gt[0, j, 4]
        gl = gt[0, j, 5]
        g6 = gt[0, j, 6]
        g7 = gt[0, j, 7]
        gabs = (jnp.abs(gx) + jnp.abs(gy) + jnp.abs(gz) + jnp.abs(gh)
                + jnp.abs(gw) + jnp.abs(gl) + jnp.abs(g6) + jnp.abs(g7))
        gvalid = gabs > 0.0
        ox = jnp.maximum(
            jnp.minimum(xl_max, gx + gl * 0.5)
            - jnp.maximum(xl_min, gx - gl * 0.5), 0.0)
        oz = jnp.maximum(
            jnp.minimum(zw_max, gz + gw * 0.5)
            - jnp.maximum(zw_min, gz - gw * 0.5), 0.0)
        oh = jnp.maximum(
            jnp.minimum(Y, gy) - jnp.maximum(ymh, gy - gh), 0.0)
        inter = ox * oz * oh
        vb = gh * gw * gl
        iou = inter / jnp.maximum(va + vb - inter, 1e-6)
        iou = jnp.where(gvalid, iou, -1.0)
        upd = iou > max_ov
        return (jnp.where(upd, iou, max_ov),
                jnp.where(upd, j, gt_asgn))

    init = (jnp.full((MR, MC), neg_inf, jnp.float32),
            jnp.zeros((MR, MC), jnp.int32))
    max_ov, gt_asgn = lax.fori_loop(0, N, iou_step, init)

    ri = lax.broadcasted_iota(jnp.int32, (MR, MC), 0)
    ci = lax.broadcasted_iota(jnp.int32, (MR, MC), 1)
    idxp = ri * MC + ci
    validm = idxp < M
    big = jnp.int32(2 ** 30)
    ga_s[...] = gt_asgn
    lanei = lax.broadcasted_iota(jnp.int32, (1, MC), 1)

    def make_select(sign):
        def select_step(k, _):
            kv = key_s[...]
            mval = jnp.max(kv)
            pidx = jnp.min(jnp.where(kv == mval, idxp, big))
            row = pidx // MC
            col = pidx % MC
            cm = lanei == col
            cmf = cm.astype(jnp.float32)

            def ext(ref):
                return jnp.sum(ref[0, pl.ds(row, 1), :] * cmf)

            seliou[0, 0, k] = sign * mval
            selbox[0, k, 0] = ext(rx)
            selbox[0, k, 1] = ext(ry)
            selbox[0, k, 2] = ext(rz)
            selbox[0, k, 3] = ext(rh)
            selbox[0, k, 4] = ext(rw)
            selbox[0, k, 5] = ext(rl)
            selbox[0, k, 6] = ext(rr)
            selbox[0, k, 7] = jnp.float32(0.0)
            ga = jnp.sum(ga_s[pl.ds(row, 1), :] * cm.astype(jnp.int32))
            selgt[0, k, 0] = gt[0, ga, 0]
            selgt[0, k, 1] = gt[0, ga, 1]
            selgt[0, k, 2] = gt[0, ga, 2]
            selgt[0, k, 3] = gt[0, ga, 3]
            selgt[0, k, 4] = gt[0, ga, 4]
            selgt[0, k, 5] = gt[0, ga, 5]
            selgt[0, k, 6] = gt[0, ga, 6]
            selgt[0, k, 7] = gt[0, ga, 7]
            rowv = key_s[pl.ds(row, 1), :]
            key_s[pl.ds(row, 1), :] = jnp.where(cm, neg_inf, rowv)
            return 0
        return select_step

    key_s[...] = jnp.where(validm, max_ov, neg_inf)
    lax.fori_loop(0, FG_PER_IMAGE, make_select(jnp.float32(1.0)), 0)

    key_s[...] = jnp.where(validm, -max_ov, neg_inf)

    sel_bg = make_select(jnp.float32(-1.0))

    def select_step_bg(k, _):
        return sel_bg(FG_PER_IMAGE + k, _)

    lax.fori_loop(0, BG_PER_IMAGE, select_step_bg, 0)


def _run_k1(planes, gt):
    plane_spec = pl.BlockSpec((1, MR, MC), lambda b: (b, 0, 0))
    smem = pltpu.SMEM
    return pl.pallas_call(
        _k1_body,
        grid=(B,),
        in_specs=[plane_spec] * 7 + [
            pl.BlockSpec((1, N, 8), lambda b: (b, 0, 0), memory_space=smem)],
        out_specs=[
            pl.BlockSpec((1, ROI_PER_IMAGE, 8), lambda b: (b, 0, 0),
                         memory_space=smem),
            pl.BlockSpec((1, ROI_PER_IMAGE, 8), lambda b: (b, 0, 0),
                         memory_space=smem),
            pl.BlockSpec((1, 1, ROI_PER_IMAGE), lambda b: (b, 0, 0),
                         memory_space=smem),
        ],
        out_shape=[
            jax.ShapeDtypeStruct((B, ROI_PER_IMAGE, 8), jnp.float32),
            jax.ShapeDtypeStruct((B, ROI_PER_IMAGE, 8), jnp.float32),
            jax.ShapeDtypeStruct((B, 1, ROI_PER_IMAGE), jnp.float32),
        ],
        scratch_shapes=[
            pltpu.VMEM((MR, MC), jnp.float32),
            pltpu.VMEM((MR, MC), jnp.int32),
        ],
    )(*planes, gt)


K2_BATCH = 8


def _k2_body(px, py, pz, selbox, trig, pos, nin):
    X = px[0]
    Y = py[0]
    Z = pz[0]
    ri = lax.broadcasted_iota(jnp.int32, (PR, PC), 0)
    ci = lax.broadcasted_iota(jnp.int32, (PR, PC), 1)
    tu = (ri <= ci).astype(jnp.float32)
    tl = (ci < ri).astype(jnp.float32)
    i1 = (ri * PC + ci + 1).astype(jnp.float32)

    for r in range(K2_BATCH):
        t = pl.program_id(0) * K2_BATCH + r
        bx = selbox[t, 0]
        by = selbox[t, 1]
        bz = selbox[t, 2]
        bh = selbox[t, 3]
        bw = selbox[t, 4]
        bl = selbox[t, 5]
        cc = trig[t, 0]
        ss = trig[t, 1]

        dx = X - bx
        dy = Y - by
        dz = Z - bz
        lx = cc * dx + ss * dz
        lz = -ss * dx + cc * dz
        in_box = ((jnp.abs(lx) < bl * 0.5 + EW)
                  & (jnp.abs(lz) < bw * 0.5 + EW)
                  & (dy > -bh - EW) & (dy < EW))
        m = in_box.astype(jnp.float32)

        inrow = jnp.dot(m, tu, preferred_element_type=jnp.float32)
        rowsum = inrow[:, PC - 1 : PC]
        prevrows = jnp.dot(tl, rowsum, preferred_element_type=jnp.float32)
        c_in = inrow + prevrows
        n_in = jnp.sum(m)
        c_out = i1 - c_in
        posf = jnp.where(in_box, c_in - 1.0, n_in + c_out - 1.0)
        pos[r] = posf.astype(jnp.int32)
        nin[r, 0, 0] = n_in.astype(jnp.int32)


def _run_k2(xyz_planes, selbox256, trig256):
    plane_spec = pl.BlockSpec(
        (1, PR, PC), lambda t: (t * K2_BATCH // ROI_PER_IMAGE, 0, 0))
    smem = pltpu.SMEM
    return pl.pallas_call(
        _k2_body,
        grid=(NPAIR // K2_BATCH,),
        in_specs=[plane_spec] * 3 + [
            pl.BlockSpec(memory_space=smem),
            pl.BlockSpec(memory_space=smem),
        ],
        out_specs=[
            pl.BlockSpec((K2_BATCH, PR, PC), lambda t: (t, 0, 0)),
            pl.BlockSpec((K2_BATCH, 1, 1), lambda t: (t, 0, 0),
                         memory_space=smem),
        ],
        out_shape=[
            jax.ShapeDtypeStruct((NPAIR, PR, PC), jnp.int32),
            jax.ShapeDtypeStruct((NPAIR, 1, 1), jnp.int32),
        ],
    )(*xyz_planes, selbox256, trig256)


def _k4_body(selbox, selgt, seliou, nin, trig, gtr, cls, reg):
    bx = selbox[:, 0:1]
    by = selbox[:, 1:2]
    bz = selbox[:, 2:3]
    cc = trig[:, 0:1]
    ss = trig[:, 1:2]
    roi_ry = trig[:, 2:3]

    gx = selgt[:, 0:1]
    gy = selgt[:, 1:2]
    gz = selgt[:, 2:3]
    dx = gx - bx
    dy = gy - by
    dz = gz - bz
    cx = cc * dx + ss * dz
    cz = -ss * dx + cc * dz
    g_ry = selgt[:, 6:7] - roi_ry
    gtr[...] = jnp.concatenate(
        [cx, dy, cz, selgt[:, 3:4], selgt[:, 4:5], selgt[:, 5:6], g_ry,
         selgt[:, 7:8]], axis=1)

    iou = seliou[:, 0:1]
    empty = nin[:, 0:1] == 0
    reg[...] = ((iou > REG_FG_THRESH) & jnp.logical_not(empty)).astype(jnp.int32)
    c0 = (iou > CLS_FG_THRESH).astype(jnp.int32)
    invalid = (iou > CLS_BG_THRESH) & (iou < CLS_FG_THRESH)
    c0 = jnp.where(empty | invalid, -1, c0)
    gcls = selgt[:, 7:8].astype(jnp.int32)
    cls[...] = jnp.where(c0 == 1, gcls, c0)


def _run_k4(selbox256, selgt256, seliou256, nin, trig256):
    return pl.pallas_call(
        _k4_body,
        out_shape=[
            jax.ShapeDtypeStruct((NPAIR, 8), jnp.float32),
            jax.ShapeDtypeStruct((NPAIR, 1), jnp.int32),
            jax.ShapeDtypeStruct((NPAIR, 1), jnp.int32),
        ],
    )(selbox256, selgt256, seliou256, nin, trig256)


PAIRS_PER_W = NPAIR // 32
ROWCHUNK = 64


def _sc_body(nc, pos_hbm, feat_hbm, xh, yh, zh, sh, dh, params_hbm,
             out_f128, out_sdt, out_xyzt,
             xv, yv, zv, sv, dv, pos_v, order_v, ordg_v, rows_a, rows_b,
             sdt_v, xyzt_v, pv, sg0, sg1, so0, so1):
    wid = lax.axis_index("s") * nc + lax.axis_index("c")
    scene = wid // (32 // B)
    sbase = scene * P

    pltpu.sync_copy(xh.at[pl.ds(sbase, P)], xv)
    pltpu.sync_copy(yh.at[pl.ds(sbase, P)], yv)
    pltpu.sync_copy(zh.at[pl.ds(sbase, P)], zv)
    pltpu.sync_copy(sh.at[pl.ds(sbase, P)], sv)
    pltpu.sync_copy(dh.at[pl.ds(sbase, P)], dv)

    lane = lax.iota(jnp.int32, 16)

    for j in range(PAIRS_PER_W):
        pair = wid * PAIRS_PER_W + j

        pltpu.sync_copy(pos_hbm.at[pl.ds(pair * P, P)], pos_v)
        pltpu.sync_copy(params_hbm.at[pl.ds(pair * 128, 128)], pv)

        def scatter_step(i, _):
            for u in range(8):
                off = i * 128 + u * 16
                p16 = pos_v[pl.ds(off, 16)]
                plsc.store_scatter(order_v, [p16], lane + off,
                                   mask=p16 < NUM_POINTS)
            return 0
        lax.fori_loop(0, P // 128, scatter_step, 0)

        def add_step(i, _):
            ordg_v[pl.ds(i * 16, 16)] = order_v[pl.ds(i * 16, 16)] + sbase
            return 0
        lax.fori_loop(0, NUM_POINTS // 16, add_step, 0)

        bufs = (rows_a, rows_b)
        sgs = (sg0, sg1)
        sos = (so0, so1)
        nchunk = NUM_POINTS // ROWCHUNK

        def g_start(c):
            return pltpu.async_copy(
                feat_hbm.at[ordg_v.at[pl.ds(c * ROWCHUNK, ROWCHUNK)]],
                bufs[c % 2], sgs[c % 2])

        def o_start(c):
            return pltpu.async_copy(
                bufs[c % 2],
                out_f128.at[pair, pl.ds(c * ROWCHUNK, ROWCHUNK)],
                sos[c % 2])

        hg = {0: g_start(0)}
        ho = {}
        for c in range(nchunk):
            if c + 1 < nchunk:
                if c >= 1:
                    ho[c - 1].wait()
                hg[c + 1] = g_start(c + 1)
            hg[c].wait()
            ho[c] = o_start(c)
        ho[nchunk - 2].wait()
        ho[nchunk - 1].wait()

        cc = pv[pl.ds(0, 16)]
        ss = pv[pl.ds(16, 16)]
        bx = pv[pl.ds(32, 16)]
        by = pv[pl.ds(48, 16)]
        bz = pv[pl.ds(64, 16)]

        def rot_step(i, _):
            o16 = order_v[pl.ds(i * 16, 16)]
            xs = plsc.load_gather(xv, [o16])
            ys = plsc.load_gather(yv, [o16])
            zs = plsc.load_gather(zv, [o16])
            sg = plsc.load_gather(sv, [o16])
            dp = plsc.load_gather(dv, [o16])
            dx = xs - bx
            dyv = ys - by
            dz = zs - bz
            rxv = cc * dx + ss * dz
            rzv = -ss * dx + cc * dz
            xyzt_v[pl.ds(i * 16, 16)] = rxv
            xyzt_v[pl.ds(NUM_POINTS + i * 16, 16)] = dyv
            xyzt_v[pl.ds(2 * NUM_POINTS + i * 16, 16)] = rzv
            sdt_v[pl.ds(i * 16, 16)] = sg
            sdt_v[pl.ds(NUM_POINTS + i * 16, 16)] = dp
            return 0
        lax.fori_loop(0, NUM_POINTS // 16, rot_step, 0)

        pltpu.sync_copy(sdt_v, out_sdt.at[pl.ds(pair * 2 * NUM_POINTS,
                                                2 * NUM_POINTS)])
        pltpu.sync_copy(xyzt_v, out_xyzt.at[pl.ds(pair * 3 * NUM_POINTS,
                                                  3 * NUM_POINTS)])


def _run_sc(pos_flat, feat128, planes5, params_flat):
    info = plsc.get_sparse_core_info()
    nc = info.num_cores
    mesh = plsc.VectorSubcoreMesh(core_axis_name="c", subcore_axis_name="s")
    f = pl.kernel(
        functools.partial(_sc_body, nc),
        out_type=[
            jax.ShapeDtypeStruct((NPAIR, NUM_POINTS, 128), jnp.float32),
            jax.ShapeDtypeStruct((NPAIR * 2 * NUM_POINTS,), jnp.float32),
            jax.ShapeDtypeStruct((NPAIR * 3 * NUM_POINTS,), jnp.float32),
        ],
        mesh=mesh,
        scratch_types=[
            pltpu.VMEM((P,), jnp.float32),
            pltpu.VMEM((P,), jnp.float32),
            pltpu.VMEM((P,), jnp.float32),
            pltpu.VMEM((P,), jnp.float32),
            pltpu.VMEM((P,), jnp.float32),
            pltpu.VMEM((P,), jnp.int32),
            pltpu.VMEM((NUM_POINTS,), jnp.int32),
            pltpu.VMEM((NUM_POINTS,), jnp.int32),
            pltpu.VMEM((ROWCHUNK, 128), jnp.float32),
            pltpu.VMEM((ROWCHUNK, 128), jnp.float32),
            pltpu.VMEM((2 * NUM_POINTS,), jnp.float32),
            pltpu.VMEM((3 * NUM_POINTS,), jnp.float32),
            pltpu.VMEM((128,), jnp.float32),
            pltpu.SemaphoreType.DMA,
            pltpu.SemaphoreType.DMA,
            pltpu.SemaphoreType.DMA,
            pltpu.SemaphoreType.DMA,
        ],
        compiler_params=pltpu.CompilerParams(
            needs_layout_passes=False, use_tc_tiling_on_sc=True),
    )
    return f(pos_flat, feat128, *planes5, params_flat)


def kernel(roi_boxes3d, gt_boxes3d, rpn_xyz, rpn_features, seg_mask, pts_depth):
    roi_p = jnp.pad(roi_boxes3d, ((0, 0), (0, MP - M), (0, 0)))
    planes = [roi_p[..., i].reshape(B, MR, MC) for i in range(7)]

    selbox, selgt, seliou = _run_k1(planes, gt_boxes3d)
    selbox256 = selbox.reshape(NPAIR, 8)
    selgt256 = selgt.reshape(NPAIR, 8)
    seliou256 = seliou.reshape(NPAIR, 1)

    ry_sel = selbox256[:, 6]
    two_pi = jnp.float32(2.0 * np.pi)
    roi_ry = ry_sel % two_pi
    trig256 = jnp.stack(
        [jnp.cos(ry_sel), jnp.sin(ry_sel), roi_ry,
         jnp.zeros_like(ry_sel)], axis=1)

    xyz_planes = [rpn_xyz[..., i].reshape(B, PR, PC) for i in range(3)]
    pos, nin = _run_k2(xyz_planes, selbox256, trig256)
    pos = pos.reshape(NPAIR, P)

    gtr, cls, reg = _run_k4(selbox256, selgt256, seliou256,
                            nin.reshape(NPAIR, 1), trig256)

    depth = pts_depth / 70.0 - 0.5
    feat128 = rpn_features.reshape(B * P, 128)
    planes5 = [rpn_xyz[..., 0].reshape(-1), rpn_xyz[..., 1].reshape(-1),
               rpn_xyz[..., 2].reshape(-1), seg_mask.reshape(-1),
               depth.reshape(-1)]
    params_flat = jnp.broadcast_to(
        jnp.concatenate([trig256[:, 0:2], selbox256[:, 0:3],
                         jnp.zeros((NPAIR, 3), jnp.float32)], axis=1)[:, :, None],
        (NPAIR, 8, 16)).reshape(-1)

    out_f128, out_sdt, out_xyzt = _run_sc(
        pos.reshape(-1), feat128, planes5, params_flat)
    sampled_pts = out_xyzt.reshape(NPAIR, 3, NUM_POINTS).transpose(0, 2, 1)
    p_feat = jnp.concatenate(
        [out_sdt.reshape(NPAIR, 2, NUM_POINTS).transpose(0, 2, 1), out_f128],
        axis=2)

    return (sampled_pts,
            p_feat,
            cls.reshape(-1),
            reg.reshape(-1),
            gtr,
            seliou.reshape(-1),
            selbox256[:, :7])
